# baseline (speedup 1.0000x reference)
import functools

import jax
import jax.numpy as jnp
from jax import lax
from jax.experimental import pallas as pl
from jax.experimental.pallas import tpu as pltpu
from jax.experimental.pallas import tpu_sc as plsc

N = 50000
E = 800000
ATOM = 10
H = 128
LAT = 128

NC = 2
NS = 16
NW = NC * NS

C = 64
NCHUNK = E // C
NBUF = 7
CP = 4
CW = H // CP

BN = 1024
NBLK = 49
PAD = NBLK * BN
ZROWS = PAD // NS


def _relu(v):
    return jnp.maximum(v, 0.0)


def _dot(a, b):
    return jnp.dot(a, b, preferred_element_type=jnp.float32)


def _embed_body(x_ref, We, be, WA, WB, bm, h_ref, hs_ref, hd_ref):
    h = _dot(x_ref[...], We[...]) + be[...]
    h_ref[...] = h
    hs_ref[...] = _dot(h, WA[...])
    hd_ref[...] = _dot(h, WB[...]) + bm[...]


def _upd_body(h_ref, p_ref, Wu, bu, WA, WB, bm, h_out, hs_ref, hd_ref):
    agg = p_ref[0] + p_ref[1]
    h = _relu(h_ref[...] + _dot(agg, Wu[...]) + bu[...])
    h_out[...] = h
    hs_ref[...] = _dot(h, WA[...])
    hd_ref[...] = _dot(h, WB[...]) + bm[...]


def _fin_body(h_ref, p_ref, Wu, bu, Wo, bo, z_ref):
    agg = p_ref[0] + p_ref[1]
    h = _relu(h_ref[...] + _dot(agg, Wu[...]) + bu[...])
    z_ref[...] = _dot(h, Wo[...]) + bo[...]


def _row_spec():
    return pl.BlockSpec((BN, H), lambda i: (i, 0))


def _part_spec():
    return pl.BlockSpec((2, BN, H), lambda i: (0, i, 0))


def _full(shape):
    return pl.BlockSpec(shape, lambda i: (0,) * len(shape))


_OUTH = [jax.ShapeDtypeStruct((N, H), jnp.float32),
         jax.ShapeDtypeStruct((PAD, H), jnp.float32),
         jax.ShapeDtypeStruct((PAD, H), jnp.float32)]

_embed_call = pl.pallas_call(
    _embed_body,
    grid=(NBLK,),
    in_specs=[
        pl.BlockSpec((BN, ATOM), lambda i: (i, 0)),
        _full((ATOM, H)), _full((1, H)), _full((H, H)), _full((H, H)),
        _full((1, H)),
    ],
    out_specs=[_row_spec()] * 3,
    out_shape=_OUTH,
)

_upd_call = pl.pallas_call(
    _upd_body,
    grid=(NBLK,),
    in_specs=[
        _row_spec(), _part_spec(),
        _full((H, H)), _full((1, H)), _full((H, H)), _full((H, H)),
        _full((1, H)),
    ],
    out_specs=[_row_spec()] * 3,
    out_shape=_OUTH,
)

_fin_call = pl.pallas_call(
    _fin_body,
    grid=(NBLK,),
    in_specs=[
        _row_spec(), _part_spec(),
        _full((H, H)), _full((1, H)), _full((H, LAT)), _full((1, LAT)),
    ],
    out_specs=_row_spec(),
    out_shape=jax.ShapeDtypeStruct((N, LAT), jnp.float32),
)


_sc_mesh = plsc.VectorSubcoreMesh(core_axis_name="c", subcore_axis_name="s")


@functools.partial(
    pl.kernel,
    out_type=jax.ShapeDtypeStruct((2, PAD, CP, CW), jnp.float32),
    mesh=_sc_mesh,
    compiler_params=pltpu.CompilerParams(use_tc_tiling_on_sc=False),
    scratch_types=[
        [pltpu.VMEM((C,), jnp.int32) for _ in range(NBUF)],
        [pltpu.VMEM((C,), jnp.int32) for _ in range(NBUF)],
        [pltpu.VMEM((C,), jnp.int32) for _ in range(NBUF)],
        [pltpu.VMEM((C, CW), jnp.float32) for _ in range(NBUF)],
        [pltpu.VMEM((C, CW), jnp.float32) for _ in range(NBUF)],
        pltpu.VMEM_SHARED((PAD, CW), jnp.float32),
        [pltpu.SemaphoreType.DMA for _ in range(NBUF)],
        [pltpu.SemaphoreType.DMA for _ in range(NBUF)],
        [pltpu.SemaphoreType.DMA for _ in range(NBUF)],
        [pltpu.SemaphoreType.DMA for _ in range(NBUF)],
    ],
)
def _edge_kernel(hs_hbm, hd_hbm, src_hbm, dst_hbm, agg_hbm,
                 sidx, didx, dgix, bufA, bufB, acc, semI, semA, semB, semS):
    c = lax.axis_index("c")
    s = lax.axis_index("s")
    wid = c * NS + s
    base_trips = NCHUNK // NW
    ntrip = jnp.where(wid < NCHUNK - base_trips * NW, base_trips + 1,
                      base_trips)

    for p in range(CP):
        def _zero_buf(r, carry):
            for k in range(CW // 16):
                bufA[0][r, pl.ds(k * 16, 16)] = jnp.zeros((16,), jnp.float32)
            return carry
        lax.fori_loop(0, C, _zero_buf, 0)
        zbase = s * ZROWS
        for t in range(ZROWS // C):
            pltpu.sync_copy(bufA[0], acc.at[pl.ds(zbase + t * C, C)])
        rem = ZROWS - (ZROWS // C) * C
        if rem:
            pltpu.sync_copy(bufA[0].at[pl.ds(0, rem)],
                            acc.at[pl.ds(zbase + (ZROWS // C) * C, rem)])
        plsc.subcore_barrier()

        def _slots(g, carry, p=p):
            for bb in range(NBUF):
                j = g * NBUF + bb
                bY = (bb - 1) % NBUF
                bZ = (bb - 3) % NBUF

                @pl.when((j >= NBUF) & (j - NBUF < ntrip))
                def _drain(bb=bb):
                    pltpu.make_async_copy(hs_hbm.at[pl.ds(0, C)], bufA[bb],
                                          semS[bb]).wait()

                @pl.when(j < ntrip)
                def _fire_idx(j=j, bb=bb):
                    eb = (wid + j * NW) * C
                    pltpu.async_copy(src_hbm.at[pl.ds(eb, C)], sidx[bb],
                                     semI[bb])
                    pltpu.async_copy(dst_hbm.at[pl.ds(eb, C)], didx[bb],
                                     semI[bb])

                @pl.when((j >= 1) & (j - 1 < ntrip))
                def _fire_gather(bY=bY, p=p):
                    pltpu.make_async_copy(src_hbm.at[pl.ds(0, C)],
                                          sidx[bY], semI[bY]).wait()
                    pltpu.make_async_copy(src_hbm.at[pl.ds(0, C)],
                                          didx[bY], semI[bY]).wait()
                    for k in range(C // 16):
                        sl = pl.ds(k * 16, 16)
                        sidx[bY][sl] = sidx[bY][sl] * CP + p
                        dgix[bY][sl] = didx[bY][sl] * CP + p
                    pltpu.async_copy(hs_hbm.at[sidx[bY]], bufA[bY], semA[bY])
                    pltpu.async_copy(hd_hbm.at[dgix[bY]], bufB[bY], semB[bY])

                @pl.when((j >= 3) & (j - 3 < ntrip))
                def _compute(bZ=bZ):
                    pltpu.make_async_copy(hs_hbm.at[pl.ds(0, C)], bufA[bZ],
                                          semA[bZ]).wait()
                    pltpu.make_async_copy(hs_hbm.at[pl.ds(0, C)], bufB[bZ],
                                          semB[bZ]).wait()

                    def _rows(rw, rc):
                        for k in range(CW // 16):
                            a = bufA[bZ][rw, pl.ds(k * 16, 16)]
                            b = bufB[bZ][rw, pl.ds(k * 16, 16)]
                            bufA[bZ][rw, pl.ds(k * 16, 16)] = _relu(a + b)
                        return rc
                    lax.fori_loop(0, C, _rows, 0)
                    pltpu.async_copy(bufA[bZ], acc.at[didx[bZ]], semS[bZ],
                                     add=True)
            return carry
        gmax = (ntrip + 2 * NBUF - 1) // NBUF
        lax.fori_loop(0, gmax, _slots, 0)
        plsc.subcore_barrier()

        pltpu.sync_copy(acc.at[pl.ds(zbase, ZROWS)],
                        agg_hbm.at[c, pl.ds(zbase, ZROWS), p])
        plsc.subcore_barrier()


def kernel(x, W_embed, b_embed, Wm0, bm0, Wu0, bu0, Wm1, bm1, Wu1, bu1,
           W_out, b_out, edge_index):
    src = edge_index[0]
    dst = edge_index[1]
    be = b_embed.reshape(1, H)
    bm0r = bm0.reshape(1, H)
    bm1r = bm1.reshape(1, H)
    bu0r = bu0.reshape(1, H)
    bu1r = bu1.reshape(1, H)
    bor = b_out.reshape(1, LAT)

    h0, hs0, hd0 = _embed_call(x, W_embed, be, Wm0[:H], Wm0[H:], bm0r)
    agg0 = _edge_kernel(hs0.reshape(CP * PAD, CW), hd0.reshape(CP * PAD, CW),
                        src, dst)
    h1, hs1, hd1 = _upd_call(h0, agg0.reshape(2, PAD, H), Wu0, bu0r,
                             Wm1[:H], Wm1[H:], bm1r)
    agg1 = _edge_kernel(hs1.reshape(CP * PAD, CW), hd1.reshape(CP * PAD, CW),
                        src, dst)
    z = _fin_call(h1, agg1.reshape(2, PAD, H), Wu1, bu1r, W_out, bor)
    return z


# speedup vs baseline: 1.0022x; 1.0022x over previous
import functools

import jax
import jax.numpy as jnp
from jax import lax
from jax.experimental import pallas as pl
from jax.experimental.pallas import tpu as pltpu
from jax.experimental.pallas import tpu_sc as plsc

N = 50000
E = 800000
ATOM = 10
H = 128
LAT = 128

NC = 2
NS = 16
NW = NC * NS

C = 64
NCHUNK = E // C
NBUF = 7
CP = 4
CW = H // CP

BN = 1024
NBLK = 49
PAD = NBLK * BN
ZROWS = PAD // NS


def _relu(v):
    return jnp.maximum(v, 0.0)


def _dot(a, b):
    return jnp.dot(a, b, preferred_element_type=jnp.float32)


def _embed_body(x_ref, We, be, WA, WB, bm, h_ref, hs_ref, hd_ref):
    h = _dot(x_ref[...], We[...]) + be[...]
    h_ref[...] = h
    hs_ref[...] = _dot(h, WA[...])
    hd_ref[...] = _dot(h, WB[...]) + bm[...]


def _upd_body(h_ref, p_ref, Wu, bu, WA, WB, bm, h_out, hs_ref, hd_ref):
    agg = p_ref[0] + p_ref[1]
    h = _relu(h_ref[...] + _dot(agg, Wu[...]) + bu[...])
    h_out[...] = h
    hs_ref[...] = _dot(h, WA[...])
    hd_ref[...] = _dot(h, WB[...]) + bm[...]


def _fin_body(h_ref, p_ref, Wu, bu, Wo, bo, z_ref):
    agg = p_ref[0] + p_ref[1]
    h = _relu(h_ref[...] + _dot(agg, Wu[...]) + bu[...])
    z_ref[...] = _dot(h, Wo[...]) + bo[...]


def _row_spec():
    return pl.BlockSpec((BN, H), lambda i: (i, 0))


def _part_spec():
    return pl.BlockSpec((2, BN, H), lambda i: (0, i, 0))


def _full(shape):
    return pl.BlockSpec(shape, lambda i: (0,) * len(shape))


_OUTH = [jax.ShapeDtypeStruct((N, H), jnp.float32),
         jax.ShapeDtypeStruct((PAD, H), jnp.float32),
         jax.ShapeDtypeStruct((PAD, H), jnp.float32)]

_embed_call = pl.pallas_call(
    _embed_body,
    grid=(NBLK,),
    in_specs=[
        pl.BlockSpec((BN, ATOM), lambda i: (i, 0)),
        _full((ATOM, H)), _full((1, H)), _full((H, H)), _full((H, H)),
        _full((1, H)),
    ],
    out_specs=[_row_spec()] * 3,
    out_shape=_OUTH,
)

_upd_call = pl.pallas_call(
    _upd_body,
    grid=(NBLK,),
    in_specs=[
        _row_spec(), _part_spec(),
        _full((H, H)), _full((1, H)), _full((H, H)), _full((H, H)),
        _full((1, H)),
    ],
    out_specs=[_row_spec()] * 3,
    out_shape=_OUTH,
)

_fin_call = pl.pallas_call(
    _fin_body,
    grid=(NBLK,),
    in_specs=[
        _row_spec(), _part_spec(),
        _full((H, H)), _full((1, H)), _full((H, LAT)), _full((1, LAT)),
    ],
    out_specs=_row_spec(),
    out_shape=jax.ShapeDtypeStruct((N, LAT), jnp.float32),
)


_sc_mesh = plsc.VectorSubcoreMesh(core_axis_name="c", subcore_axis_name="s")


@functools.partial(
    pl.kernel,
    out_type=jax.ShapeDtypeStruct((2, PAD, CP, CW), jnp.float32),
    mesh=_sc_mesh,
    compiler_params=pltpu.CompilerParams(use_tc_tiling_on_sc=False),
    scratch_types=[
        [pltpu.VMEM((C,), jnp.int32) for _ in range(NBUF)],
        [pltpu.VMEM((C,), jnp.int32) for _ in range(NBUF)],
        [pltpu.VMEM((C,), jnp.int32) for _ in range(NBUF)],
        [pltpu.VMEM((C, CW), jnp.float32) for _ in range(NBUF)],
        [pltpu.VMEM((C, CW), jnp.float32) for _ in range(NBUF)],
        pltpu.VMEM_SHARED((PAD, CW), jnp.float32),
        [pltpu.SemaphoreType.DMA for _ in range(NBUF)],
        [pltpu.SemaphoreType.DMA for _ in range(NBUF)],
        [pltpu.SemaphoreType.DMA for _ in range(NBUF)],
        [pltpu.SemaphoreType.DMA for _ in range(NBUF)],
    ],
)
def _edge_kernel(hs_hbm, hd_hbm, src_hbm, dst_hbm, agg_hbm,
                 sidx, didx, dgix, bufA, bufB, acc, semI, semA, semB, semS):
    c = lax.axis_index("c")
    s = lax.axis_index("s")
    wid = c * NS + s
    base_trips = NCHUNK // NW
    ntrip = jnp.where(wid < NCHUNK - base_trips * NW, base_trips + 1,
                      base_trips)

    for p in range(CP):
        def _zero_buf(r, carry):
            for k in range(CW // 16):
                bufA[0][r, pl.ds(k * 16, 16)] = jnp.zeros((16,), jnp.float32)
            return carry
        lax.fori_loop(0, C, _zero_buf, 0)
        zbase = s * ZROWS
        for t in range(ZROWS // C):
            pltpu.sync_copy(bufA[0], acc.at[pl.ds(zbase + t * C, C)])
        rem = ZROWS - (ZROWS // C) * C
        if rem:
            pltpu.sync_copy(bufA[0].at[pl.ds(0, rem)],
                            acc.at[pl.ds(zbase + (ZROWS // C) * C, rem)])
        plsc.subcore_barrier()

        def _slots(g, carry, p=p):
            for bb in range(NBUF):
                j = g * NBUF + bb
                bY = (bb - 1) % NBUF
                bZ = (bb - 4) % NBUF

                @pl.when((j >= NBUF) & (j - NBUF < ntrip))
                def _drain(bb=bb):
                    pltpu.make_async_copy(hs_hbm.at[pl.ds(0, C)], bufA[bb],
                                          semS[bb]).wait()

                @pl.when(j < ntrip)
                def _fire_idx(j=j, bb=bb):
                    eb = (wid + j * NW) * C
                    pltpu.async_copy(src_hbm.at[pl.ds(eb, C)], sidx[bb],
                                     semI[bb])
                    pltpu.async_copy(dst_hbm.at[pl.ds(eb, C)], didx[bb],
                                     semI[bb])

                @pl.when((j >= 1) & (j - 1 < ntrip))
                def _fire_gather(bY=bY, p=p):
                    pltpu.make_async_copy(src_hbm.at[pl.ds(0, C)],
                                          sidx[bY], semI[bY]).wait()
                    pltpu.make_async_copy(src_hbm.at[pl.ds(0, C)],
                                          didx[bY], semI[bY]).wait()
                    for k in range(C // 16):
                        sl = pl.ds(k * 16, 16)
                        sidx[bY][sl] = sidx[bY][sl] * CP + p
                        dgix[bY][sl] = didx[bY][sl] * CP + p
                    pltpu.async_copy(hs_hbm.at[sidx[bY]], bufA[bY], semA[bY])
                    pltpu.async_copy(hd_hbm.at[dgix[bY]], bufB[bY], semB[bY])

                @pl.when((j >= 4) & (j - 4 < ntrip))
                def _compute(bZ=bZ):
                    pltpu.make_async_copy(hs_hbm.at[pl.ds(0, C)], bufA[bZ],
                                          semA[bZ]).wait()
                    pltpu.make_async_copy(hs_hbm.at[pl.ds(0, C)], bufB[bZ],
                                          semB[bZ]).wait()

                    def _rows(rw, rc):
                        for k in range(CW // 16):
                            a = bufA[bZ][rw, pl.ds(k * 16, 16)]
                            b = bufB[bZ][rw, pl.ds(k * 16, 16)]
                            bufA[bZ][rw, pl.ds(k * 16, 16)] = _relu(a + b)
                        return rc
                    lax.fori_loop(0, C, _rows, 0)
                    pltpu.async_copy(bufA[bZ], acc.at[didx[bZ]], semS[bZ],
                                     add=True)
            return carry
        gmax = (ntrip + 2 * NBUF - 1) // NBUF
        lax.fori_loop(0, gmax, _slots, 0)
        plsc.subcore_barrier()

        pltpu.sync_copy(acc.at[pl.ds(zbase, ZROWS)],
                        agg_hbm.at[c, pl.ds(zbase, ZROWS), p])
        plsc.subcore_barrier()


def kernel(x, W_embed, b_embed, Wm0, bm0, Wu0, bu0, Wm1, bm1, Wu1, bu1,
           W_out, b_out, edge_index):
    src = edge_index[0]
    dst = edge_index[1]
    be = b_embed.reshape(1, H)
    bm0r = bm0.reshape(1, H)
    bm1r = bm1.reshape(1, H)
    bu0r = bu0.reshape(1, H)
    bu1r = bu1.reshape(1, H)
    bor = b_out.reshape(1, LAT)

    h0, hs0, hd0 = _embed_call(x, W_embed, be, Wm0[:H], Wm0[H:], bm0r)
    agg0 = _edge_kernel(hs0.reshape(CP * PAD, CW), hd0.reshape(CP * PAD, CW),
                        src, dst)
    h1, hs1, hd1 = _upd_call(h0, agg0.reshape(2, PAD, H), Wu0, bu0r,
                             Wm1[:H], Wm1[H:], bm1r)
    agg1 = _edge_kernel(hs1.reshape(CP * PAD, CW), hd1.reshape(CP * PAD, CW),
                        src, dst)
    z = _fin_call(h1, agg1.reshape(2, PAD, H), Wu1, bu1r, W_out, bor)
    return z
